# baseline (speedup 1.0000x reference)
import functools

import jax
import jax.numpy as jnp
from jax import lax
from jax.experimental import pallas as pl
from jax.experimental.pallas import tpu as pltpu
from jax.experimental.pallas import tpu_sc as plsc

N = 10000
D = 128
DH = D // 2
E = 320000

NC = 2
NS = 16
NW = NC * NS
CHUNK = 128
EPAD = 327680
SCHUNK = 64
NCHUNK = EPAD // (NS * SCHUNK)
NBUF = 2
NPAD = 10016
STRIP = NPAD // NS
HPAD = 10240
HSTRIP = HPAD // NS
HCHUNK = EPAD // (NW * CHUNK)

_mesh = plsc.VectorSubcoreMesh(core_axis_name="c", subcore_axis_name="s")


@functools.partial(
    pl.kernel,
    out_type=jax.ShapeDtypeStruct((NC, HPAD, 16), jnp.float32),
    mesh=_mesh,
    scratch_types=[
        pltpu.VMEM((HCHUNK, CHUNK), jnp.int32),
        pltpu.VMEM((CHUNK, 16), jnp.float32),
        pltpu.VMEM_SHARED((HPAD, 16), jnp.float32),
        pltpu.SemaphoreType.DMA,
    ],
)
def _sc_hist(src_hbm, ones_hbm, zeros_hbm, out_hbm, idx_v, ones_v, hist_sh,
             sem):
    c = lax.axis_index("c")
    s = lax.axis_index("s")
    wid = c * NS + s
    pltpu.sync_copy(src_hbm.at[wid], idx_v)
    pltpu.sync_copy(ones_hbm, ones_v)
    pltpu.sync_copy(zeros_hbm.at[pl.ds(s * HSTRIP, HSTRIP)],
                    hist_sh.at[pl.ds(s * HSTRIP, HSTRIP)])
    plsc.subcore_barrier()

    @pl.loop(0, HCHUNK)
    def _(j):
        pltpu.async_copy(ones_v, hist_sh.at[idx_v.at[j]], sem, add=True)

    pltpu.make_async_copy(zeros_hbm, hist_sh, sem).wait()
    plsc.subcore_barrier()
    pltpu.sync_copy(hist_sh.at[pl.ds(s * HSTRIP, HSTRIP)],
                    out_hbm.at[c, pl.ds(s * HSTRIP, HSTRIP)])


@functools.partial(
    pl.kernel,
    out_type=jax.ShapeDtypeStruct((NC, NPAD, DH), jnp.float32),
    mesh=_mesh,
    scratch_types=[
        pltpu.VMEM((NCHUNK, SCHUNK), jnp.int32),
        pltpu.VMEM((NCHUNK, SCHUNK), jnp.int32),
        pltpu.VMEM((NBUF * SCHUNK, DH), jnp.float32),
        pltpu.VMEM_SHARED((NPAD, DH), jnp.float32),
        pltpu.VMEM_SHARED((N, DH), jnp.float32),
        [pltpu.SemaphoreType.DMA for _ in range(NBUF)],
        [pltpu.SemaphoreType.DMA for _ in range(NBUF)],
    ],
    compiler_params=pltpu.CompilerParams(use_tc_tiling_on_sc=False),
)
def _sc_spmm(src_hbm, dst_hbm, z2_hbm, out_hbm,
             si_v, di_v, bufs, w_sh, z_sh, sems, sems_s):
    c = lax.axis_index("c")
    s = lax.axis_index("s")
    pltpu.sync_copy(src_hbm.at[s], si_v)
    pltpu.sync_copy(dst_hbm.at[s], di_v)

    zrows = NBUF * SCHUNK
    @pl.loop(0, zrows)
    def _(i):
        for k in range(DH // 16):
            bufs[i, pl.ds(k * 16, 16)] = jnp.zeros((16,), jnp.float32)

    base = s * STRIP
    off = 0
    while off < STRIP:
        n = min(zrows, STRIP - off)
        pltpu.sync_copy(bufs.at[pl.ds(0, n)], w_sh.at[pl.ds(base + off, n)])
        off += n
    @pl.when(s < NS // 2)
    def _():
        pltpu.sync_copy(z2_hbm.at[c, pl.ds(s * (N // 8), N // 8)],
                        z_sh.at[pl.ds(s * (N // 8), N // 8)])

    plsc.subcore_barrier()

    zc = z_sh

    def buf(b):
        return bufs.at[pl.ds(b * SCHUNK, SCHUNK)]

    def gather_start(j, b):
        pltpu.async_copy(zc.at[di_v.at[j]], buf(b), sems[b])

    def gather_wait(j, b):
        pltpu.make_async_copy(zc.at[di_v.at[j]], buf(b), sems[b]).wait()

    def scatter_start(j, b):
        pltpu.async_copy(buf(b), w_sh.at[si_v.at[j]], sems_s[b], add=True)

    def scatter_wait(j, b):
        pltpu.make_async_copy(buf(b), w_sh.at[si_v.at[j]],
                              sems_s[b]).wait()

    for b in range(NBUF):
        gather_start(b, b)

    @pl.loop(0, NCHUNK - NBUF, step=NBUF)
    def _(i):
        for b in range(NBUF):
            gather_wait(i + b, b)
            scatter_start(i + b, b)
        for b in range(NBUF):
            scatter_wait(i + b, b)
            gather_start(i + NBUF + b, b)

    for b in range(NBUF):
        gather_wait(NCHUNK - NBUF + b, b)
        scatter_start(NCHUNK - NBUF + b, b)
    for b in range(NBUF):
        scatter_wait(NCHUNK - NBUF + b, b)

    plsc.subcore_barrier()
    pltpu.sync_copy(w_sh.at[pl.ds(s * STRIP, STRIP)],
                    out_hbm.at[c, pl.ds(s * STRIP, STRIP)])


def _deg_inv_sqrt(hist):
    h = hist[0] + hist[1]
    deg = h[0:N, 0:1] + 2.0
    return lax.rsqrt(deg)


def _tc_norm_body(x_ref, xn_ref):
    x = x_ref[...]
    mean = jnp.mean(x, axis=0, keepdims=True)
    xc = x - mean
    var = jnp.sum(xc * xc, axis=0, keepdims=True) / (N - 1)
    std = jnp.sqrt(var)
    std = jnp.where(std == 0.0, 1.0, std)
    xn_ref[...] = xc / std


def _tc_scale_body(xn_ref, hist_ref, z2_ref):
    z = _deg_inv_sqrt(hist_ref[...]) * xn_ref[...]
    z2_ref[0, :, :] = z[:, 0:DH]
    z2_ref[1, :, :] = z[:, DH:D]


def _tc_final_body(w_ref, z2_ref, hist_ref, y_ref):
    w = w_ref[...]
    z2 = z2_ref[...]
    dinv = _deg_inv_sqrt(hist_ref[...])
    ylo = dinv * (w[0, 0:N, :] + 2.0 * z2[0])
    yhi = dinv * (w[1, 0:N, :] + 2.0 * z2[1])
    y_ref[...] = jnp.concatenate([ylo, yhi], axis=1)


def kernel(x, edge_index):
    src = edge_index[0]
    dst = edge_index[1]
    pad = EPAD - E
    srcp = jnp.concatenate([src, jnp.full((pad,), N, jnp.int32)])
    dstp = jnp.concatenate([dst, jnp.zeros((pad,), jnp.int32)])
    src3h = srcp.reshape(NW, HCHUNK, CHUNK)
    src3 = srcp.reshape(NS, NCHUNK, SCHUNK)
    dst3 = dstp.reshape(NS, NCHUNK, SCHUNK)

    ones16 = jnp.concatenate(
        [jnp.ones((CHUNK, 1), jnp.float32),
         jnp.zeros((CHUNK, 15), jnp.float32)], axis=1)
    zeros16 = jnp.zeros((HPAD, 16), jnp.float32)

    hist = _sc_hist(src3h, ones16, zeros16)

    xn = pl.pallas_call(
        _tc_norm_body,
        out_shape=jax.ShapeDtypeStruct((N, D), jnp.float32),
    )(x)

    z2 = pl.pallas_call(
        _tc_scale_body,
        out_shape=jax.ShapeDtypeStruct((NC, N, DH), jnp.float32),
    )(xn, hist)

    w = _sc_spmm(src3, dst3, z2)

    y = pl.pallas_call(
        _tc_final_body,
        out_shape=jax.ShapeDtypeStruct((N, D), jnp.float32),
    )(w, z2, hist)
    return y


# speedup vs baseline: 1.0355x; 1.0355x over previous
import functools

import jax
import jax.numpy as jnp
from jax import lax
from jax.experimental import pallas as pl
from jax.experimental.pallas import tpu as pltpu
from jax.experimental.pallas import tpu_sc as plsc

N = 10000
D = 128
DH = D // 2
E = 320000

NC = 2
NS = 16
NW = NC * NS
CHUNK = 128
EPAD = 327680
SCHUNK = 64
NCHUNK = EPAD // (NS * SCHUNK)
NBUF = 2
NPAD = 10016
STRIP = NPAD // NS
HPAD = 10240
HSTRIP = HPAD // NS
HCHUNK = EPAD // (NW * CHUNK)

_mesh = plsc.VectorSubcoreMesh(core_axis_name="c", subcore_axis_name="s")


@functools.partial(
    pl.kernel,
    out_type=jax.ShapeDtypeStruct((NC, HPAD, 16), jnp.float32),
    mesh=_mesh,
    scratch_types=[
        pltpu.VMEM((HCHUNK, CHUNK), jnp.int32),
        pltpu.VMEM((CHUNK, 16), jnp.float32),
        pltpu.VMEM_SHARED((HPAD, 16), jnp.float32),
        pltpu.SemaphoreType.DMA,
    ],
)
def _sc_hist(src_hbm, ones_hbm, zeros_hbm, out_hbm, idx_v, ones_v, hist_sh,
             sem):
    c = lax.axis_index("c")
    s = lax.axis_index("s")
    wid = c * NS + s
    pltpu.sync_copy(src_hbm.at[wid], idx_v)
    pltpu.sync_copy(ones_hbm, ones_v)
    pltpu.sync_copy(zeros_hbm.at[pl.ds(s * HSTRIP, HSTRIP)],
                    hist_sh.at[pl.ds(s * HSTRIP, HSTRIP)])
    plsc.subcore_barrier()

    @pl.loop(0, HCHUNK)
    def _(j):
        pltpu.async_copy(ones_v, hist_sh.at[idx_v.at[j]], sem, add=True)

    pltpu.make_async_copy(zeros_hbm, hist_sh, sem).wait()
    plsc.subcore_barrier()
    pltpu.sync_copy(hist_sh.at[pl.ds(s * HSTRIP, HSTRIP)],
                    out_hbm.at[c, pl.ds(s * HSTRIP, HSTRIP)])


@functools.partial(
    pl.kernel,
    out_type=jax.ShapeDtypeStruct((NC, NPAD, DH), jnp.float32),
    mesh=_mesh,
    scratch_types=[
        pltpu.VMEM((NCHUNK, SCHUNK), jnp.int32),
        pltpu.VMEM((NCHUNK, SCHUNK), jnp.int32),
        pltpu.VMEM((NBUF * SCHUNK, DH), jnp.float32),
        pltpu.VMEM_SHARED((NPAD, DH), jnp.float32),
        pltpu.VMEM_SHARED((N, DH), jnp.float32),
        [pltpu.SemaphoreType.DMA for _ in range(NBUF)],
    ],
    compiler_params=pltpu.CompilerParams(use_tc_tiling_on_sc=False),
)
def _sc_spmm(src_hbm, dst_hbm, z2_hbm, out_hbm,
             si_v, di_v, bufs, w_sh, z_sh, sems):
    c = lax.axis_index("c")
    s = lax.axis_index("s")
    pltpu.sync_copy(src_hbm.at[s], si_v)
    pltpu.sync_copy(dst_hbm.at[s], di_v)

    zrows = NBUF * SCHUNK
    @pl.loop(0, zrows)
    def _(i):
        for k in range(DH // 16):
            bufs[i, pl.ds(k * 16, 16)] = jnp.zeros((16,), jnp.float32)

    base = s * STRIP
    off = 0
    while off < STRIP:
        n = min(zrows, STRIP - off)
        pltpu.sync_copy(bufs.at[pl.ds(0, n)], w_sh.at[pl.ds(base + off, n)])
        off += n
    @pl.when(s < NS // 2)
    def _():
        pltpu.sync_copy(z2_hbm.at[c, pl.ds(s * (N // 8), N // 8)],
                        z_sh.at[pl.ds(s * (N // 8), N // 8)])

    plsc.subcore_barrier()

    zc = z_sh

    def buf(b):
        return bufs.at[pl.ds(b * SCHUNK, SCHUNK)]

    def gather_start(j, b):
        pltpu.async_copy(zc.at[di_v.at[j]], buf(b), sems[b])

    def gather_wait(j, b):
        pltpu.make_async_copy(zc.at[di_v.at[j]], buf(b), sems[b]).wait()

    def scatter_sync(j, b):
        pltpu.sync_copy(buf(b), w_sh.at[si_v.at[j]], add=True)

    for b in range(NBUF):
        gather_start(b, b)

    @pl.loop(0, NCHUNK - NBUF, step=NBUF)
    def _(i):
        for b in range(NBUF):
            gather_wait(i + b, b)
            scatter_sync(i + b, b)
            gather_start(i + NBUF + b, b)

    for b in range(NBUF):
        gather_wait(NCHUNK - NBUF + b, b)
        scatter_sync(NCHUNK - NBUF + b, b)

    plsc.subcore_barrier()
    pltpu.sync_copy(w_sh.at[pl.ds(s * STRIP, STRIP)],
                    out_hbm.at[c, pl.ds(s * STRIP, STRIP)])


def _deg_inv_sqrt(hist):
    h = hist[0] + hist[1]
    deg = h[0:N, 0:1] + 2.0
    return lax.rsqrt(deg)


def _tc_norm_body(x_ref, xn_ref):
    x = x_ref[...]
    mean = jnp.mean(x, axis=0, keepdims=True)
    xc = x - mean
    var = jnp.sum(xc * xc, axis=0, keepdims=True) / (N - 1)
    std = jnp.sqrt(var)
    std = jnp.where(std == 0.0, 1.0, std)
    xn_ref[...] = xc / std


def _tc_scale_body(xn_ref, hist_ref, z2_ref):
    z = _deg_inv_sqrt(hist_ref[...]) * xn_ref[...]
    z2_ref[0, :, :] = z[:, 0:DH]
    z2_ref[1, :, :] = z[:, DH:D]


def _tc_final_body(w_ref, z2_ref, hist_ref, y_ref):
    w = w_ref[...]
    z2 = z2_ref[...]
    dinv = _deg_inv_sqrt(hist_ref[...])
    ylo = dinv * (w[0, 0:N, :] + 2.0 * z2[0])
    yhi = dinv * (w[1, 0:N, :] + 2.0 * z2[1])
    y_ref[...] = jnp.concatenate([ylo, yhi], axis=1)


def kernel(x, edge_index):
    src = edge_index[0]
    dst = edge_index[1]
    pad = EPAD - E
    srcp = jnp.concatenate([src, jnp.full((pad,), N, jnp.int32)])
    dstp = jnp.concatenate([dst, jnp.zeros((pad,), jnp.int32)])
    src3h = srcp.reshape(NW, HCHUNK, CHUNK)
    src3 = srcp.reshape(NS, NCHUNK, SCHUNK)
    dst3 = dstp.reshape(NS, NCHUNK, SCHUNK)

    ones16 = jnp.concatenate(
        [jnp.ones((CHUNK, 1), jnp.float32),
         jnp.zeros((CHUNK, 15), jnp.float32)], axis=1)
    zeros16 = jnp.zeros((HPAD, 16), jnp.float32)

    hist = _sc_hist(src3h, ones16, zeros16)

    xn = pl.pallas_call(
        _tc_norm_body,
        out_shape=jax.ShapeDtypeStruct((N, D), jnp.float32),
    )(x)

    z2 = pl.pallas_call(
        _tc_scale_body,
        out_shape=jax.ShapeDtypeStruct((NC, N, DH), jnp.float32),
    )(xn, hist)

    w = _sc_spmm(src3, dst3, z2)

    y = pl.pallas_call(
        _tc_final_body,
        out_shape=jax.ShapeDtypeStruct((N, D), jnp.float32),
    )(w, z2, hist)
    return y
